# baseline (speedup 1.0000x reference)
import functools

import jax
import jax.numpy as jnp
from jax import lax
from jax.experimental import pallas as pl
from jax.experimental.pallas import tpu as pltpu
from jax.experimental.pallas import tpu_sc as plsc

B = 8
CIN = 3
H = 512
W = 512
PATCH = 32
GH = 16
GW = 16
CODE_DIM = 256
K = 8192
PD = CIN * PATCH * PATCH
N = B * GH * GW

ROW_BLK = 256
CB_BLK = 1024
N_ROW = N // ROW_BLK
N_CB = K // CB_BLK


def _vq_body(p_ref, we_ref, cb_ref, out_ref, z_ref, min_ref, idx_ref):
    j = pl.program_id(1)

    @pl.when(j == 0)
    def _init():
        z_ref[...] = jnp.dot(p_ref[...], we_ref[...],
                             preferred_element_type=jnp.float32)
        min_ref[...] = jnp.full((ROW_BLK, 1), jnp.inf, dtype=jnp.float32)
        idx_ref[...] = jnp.zeros((ROW_BLK, 1), dtype=jnp.int32)

    cbt = cb_ref[...]
    cbn = jnp.sum(cbt * cbt, axis=0, keepdims=True)
    s = jnp.dot(z_ref[...], cbt, preferred_element_type=jnp.float32)
    d = cbn - 2.0 * s
    m = jnp.min(d, axis=1, keepdims=True)
    ii = lax.broadcasted_iota(jnp.int32, d.shape, 1)
    am = jnp.min(jnp.where(d <= m, ii, K), axis=1, keepdims=True)
    gidx = am + j * CB_BLK
    upd = m < min_ref[...]
    idx_ref[...] = jnp.where(upd, gidx, idx_ref[...])
    min_ref[...] = jnp.where(upd, m, min_ref[...])

    @pl.when(j == N_CB - 1)
    def _emit():
        out_ref[...] = idx_ref[...].reshape(1, ROW_BLK, 1)


def _vq_indices(patches, W_enc, codebook_t):
    out = pl.pallas_call(
        _vq_body,
        grid=(N_ROW, N_CB),
        in_specs=[
            pl.BlockSpec((ROW_BLK, PD), lambda i, j: (i, 0)),
            pl.BlockSpec((PD, CODE_DIM), lambda i, j: (0, 0)),
            pl.BlockSpec((CODE_DIM, CB_BLK), lambda i, j: (0, j)),
        ],
        out_specs=pl.BlockSpec((1, ROW_BLK, 1), lambda i, j: (i, 0, 0)),
        out_shape=jax.ShapeDtypeStruct((N_ROW, ROW_BLK, 1), jnp.int32),
        scratch_shapes=[
            pltpu.VMEM((ROW_BLK, CODE_DIM), jnp.float32),
            pltpu.VMEM((ROW_BLK, 1), jnp.float32),
            pltpu.VMEM((ROW_BLK, 1), jnp.int32),
        ],
        compiler_params=pltpu.CompilerParams(
            dimension_semantics=("parallel", "arbitrary"),
        ),
    )(patches, W_enc, codebook_t)
    return out.reshape(N)


NW = 32
B_PER_W = N // NW


def _sc_gather_body(cb_hbm, idx_hbm, out_hbm, idx_v, rows_v, sem):
    wid = lax.axis_index("s") * 2 + lax.axis_index("c")
    base = wid * B_PER_W
    pltpu.sync_copy(idx_hbm.at[pl.ds(base, B_PER_W)], idx_v)
    pltpu.async_copy(cb_hbm.at[idx_v], rows_v, sem).wait()
    pltpu.sync_copy(rows_v, out_hbm.at[pl.ds(base, B_PER_W)])


def _sc_gather(codebook, inds):
    k = functools.partial(
        pl.kernel,
        out_type=jax.ShapeDtypeStruct((N, CODE_DIM), jnp.float32),
        mesh=plsc.VectorSubcoreMesh(core_axis_name="c", subcore_axis_name="s"),
        scratch_types=[
            pltpu.VMEM((B_PER_W,), jnp.int32),
            pltpu.VMEM((B_PER_W, CODE_DIM), jnp.float32),
            pltpu.SemaphoreType.DMA,
        ],
    )(_sc_gather_body)
    return k(codebook, inds)


def _dec_body(hq_ref, wd_ref, out_ref):
    out_ref[...] = jnp.dot(hq_ref[...], wd_ref[...],
                           preferred_element_type=jnp.float32)


def _decode(hq, W_dec):
    return pl.pallas_call(
        _dec_body,
        grid=(N_ROW,),
        in_specs=[
            pl.BlockSpec((ROW_BLK, CODE_DIM), lambda i: (i, 0)),
            pl.BlockSpec((CODE_DIM, PD), lambda i: (0, 0)),
        ],
        out_specs=pl.BlockSpec((ROW_BLK, PD), lambda i: (i, 0)),
        out_shape=jax.ShapeDtypeStruct((N, PD), jnp.float32),
        compiler_params=pltpu.CompilerParams(
            dimension_semantics=("parallel",),
        ),
    )(hq, W_dec)


def kernel(x, W_enc, codebook, W_dec):
    p = x.reshape(B, CIN, GH, PATCH, GW, PATCH)
    p = p.transpose(0, 2, 4, 1, 3, 5).reshape(N, PD)

    inds = _vq_indices(p, W_enc, codebook.T)
    hq = _sc_gather(codebook, inds)
    op = _decode(hq, W_dec)

    out = op.reshape(B, GH, GW, CIN, PATCH, PATCH)
    out = out.transpose(0, 3, 1, 4, 2, 5).reshape(B, CIN, H, W)
    return out


# speedup vs baseline: 1.1041x; 1.1041x over previous
import functools

import jax
import jax.numpy as jnp
from jax import lax
from jax.experimental import pallas as pl
from jax.experimental.pallas import tpu as pltpu
from jax.experimental.pallas import tpu_sc as plsc

B = 8
CIN = 3
H = 512
W = 512
PATCH = 32
GH = 16
GW = 16
CODE_DIM = 256
K = 8192
PD = CIN * PATCH * PATCH
N = B * GH * GW

ROW_BLK = 256
CB_BLK = 1024
N_ROW = N // ROW_BLK
N_CB = K // CB_BLK


def _vq_body(p_ref, we_ref, cb_ref, out_ref, cn_ref):
    i = pl.program_id(0)
    z = jnp.dot(p_ref[...], we_ref[...], preferred_element_type=jnp.float32)

    @pl.when(i == 0)
    def _norms():
        for j in range(N_CB):
            cbj = cb_ref[pl.ds(j * CB_BLK, CB_BLK), :]
            cn_ref[pl.ds(j * CB_BLK, CB_BLK), :] = jnp.sum(
                cbj * cbj, axis=1, keepdims=True)

    best_m = jnp.full((1, ROW_BLK), jnp.inf, dtype=jnp.float32)
    best_i = jnp.zeros((1, ROW_BLK), dtype=jnp.int32)
    for j in range(N_CB):
        cbj = cb_ref[pl.ds(j * CB_BLK, CB_BLK), :]
        st = lax.dot_general(cbj, z, (((1,), (1,)), ((), ())),
                             preferred_element_type=jnp.float32)
        dt = cn_ref[pl.ds(j * CB_BLK, CB_BLK), :] - 2.0 * st
        m = jnp.min(dt, axis=0, keepdims=True)
        ii = lax.broadcasted_iota(jnp.int32, dt.shape, 0)
        am = jnp.min(jnp.where(dt <= m, ii, K), axis=0, keepdims=True) + j * CB_BLK
        upd = m < best_m
        best_i = jnp.where(upd, am, best_i)
        best_m = jnp.where(upd, m, best_m)
    out_ref[...] = best_i.reshape(1, 1, ROW_BLK)


def _vq_indices(patches, W_enc, codebook):
    out = pl.pallas_call(
        _vq_body,
        grid=(N_ROW,),
        in_specs=[
            pl.BlockSpec((ROW_BLK, PD), lambda i: (i, 0)),
            pl.BlockSpec((PD, CODE_DIM), lambda i: (0, 0)),
            pl.BlockSpec((K, CODE_DIM), lambda i: (0, 0)),
        ],
        out_specs=pl.BlockSpec((1, 1, ROW_BLK), lambda i: (i, 0, 0)),
        out_shape=jax.ShapeDtypeStruct((N_ROW, 1, ROW_BLK), jnp.int32),
        scratch_shapes=[
            pltpu.VMEM((K, 1), jnp.float32),
        ],
        compiler_params=pltpu.CompilerParams(
            dimension_semantics=("arbitrary",),
        ),
    )(patches, W_enc, codebook)
    return out.reshape(N)


NW = 32
B_PER_W = N // NW


def _sc_gather_body(cb_hbm, idx_hbm, out_hbm, idx_v, rows_v, sem):
    wid = lax.axis_index("s") * 2 + lax.axis_index("c")
    base = wid * B_PER_W
    pltpu.sync_copy(idx_hbm.at[pl.ds(base, B_PER_W)], idx_v)
    pltpu.async_copy(cb_hbm.at[idx_v], rows_v, sem).wait()
    pltpu.sync_copy(rows_v, out_hbm.at[pl.ds(base, B_PER_W)])


def _sc_gather(codebook, inds):
    k = functools.partial(
        pl.kernel,
        out_type=jax.ShapeDtypeStruct((N, CODE_DIM), jnp.float32),
        mesh=plsc.VectorSubcoreMesh(core_axis_name="c", subcore_axis_name="s"),
        scratch_types=[
            pltpu.VMEM((B_PER_W,), jnp.int32),
            pltpu.VMEM((B_PER_W, CODE_DIM), jnp.float32),
            pltpu.SemaphoreType.DMA,
        ],
    )(_sc_gather_body)
    return k(codebook, inds)


def _dec_body(hq_ref, wd_ref, out_ref):
    out_ref[...] = jnp.dot(hq_ref[...], wd_ref[...],
                           preferred_element_type=jnp.float32)


def _decode(hq, W_dec):
    return pl.pallas_call(
        _dec_body,
        grid=(N_ROW,),
        in_specs=[
            pl.BlockSpec((ROW_BLK, CODE_DIM), lambda i: (i, 0)),
            pl.BlockSpec((CODE_DIM, PD), lambda i: (0, 0)),
        ],
        out_specs=pl.BlockSpec((ROW_BLK, PD), lambda i: (i, 0)),
        out_shape=jax.ShapeDtypeStruct((N, PD), jnp.float32),
        compiler_params=pltpu.CompilerParams(
            dimension_semantics=("parallel",),
        ),
    )(hq, W_dec)


def kernel(x, W_enc, codebook, W_dec):
    p = x.reshape(B, CIN, GH, PATCH, GW, PATCH)
    p = p.transpose(0, 2, 4, 1, 3, 5).reshape(N, PD)

    inds = _vq_indices(p, W_enc, codebook)
    hq = _sc_gather(codebook, inds)
    op = _decode(hq, W_dec)

    out = op.reshape(B, GH, GW, CIN, PATCH, PATCH)
    out = out.transpose(0, 3, 1, 4, 2, 5).reshape(B, CIN, H, W)
    return out


# speedup vs baseline: 2.5425x; 2.3028x over previous
import functools

import jax
import jax.numpy as jnp
from jax import lax
from jax.experimental import pallas as pl
from jax.experimental.pallas import tpu as pltpu
from jax.experimental.pallas import tpu_sc as plsc

B = 8
CIN = 3
H = 512
W = 512
PATCH = 32
GH = 16
GW = 16
CODE_DIM = 256
K = 8192
PD = CIN * PATCH * PATCH
N = B * GH * GW

ROW_BLK = 256
CB_BLK = 1024
N_ROW = N // ROW_BLK
N_CB = K // CB_BLK


def _vq_body(x_ref, we_ref, cb_ref, out_ref, cn_ref):
    i = pl.program_id(0)
    x5 = x_ref[0].reshape(CIN, GH, PATCH, GW, PATCH)
    p = x5.transpose(1, 3, 0, 2, 4).reshape(GH * GW, PD)
    z = jnp.dot(p, we_ref[...], preferred_element_type=jnp.float32)

    @pl.when(i == 0)
    def _norms():
        for j in range(N_CB):
            cbj = cb_ref[pl.ds(j * CB_BLK, CB_BLK), :]
            cn_ref[pl.ds(j * CB_BLK, CB_BLK), :] = jnp.sum(
                cbj * cbj, axis=1, keepdims=True)

    best_m = jnp.full((1, ROW_BLK), jnp.inf, dtype=jnp.float32)
    best_i = jnp.zeros((1, ROW_BLK), dtype=jnp.int32)
    for j in range(N_CB):
        cbj = cb_ref[pl.ds(j * CB_BLK, CB_BLK), :]
        st = lax.dot_general(cbj, z, (((1,), (1,)), ((), ())),
                             preferred_element_type=jnp.float32)
        dt = cn_ref[pl.ds(j * CB_BLK, CB_BLK), :] - 2.0 * st
        m = jnp.min(dt, axis=0, keepdims=True)
        ii = lax.broadcasted_iota(jnp.int32, dt.shape, 0)
        am = jnp.min(jnp.where(dt <= m, ii, K), axis=0, keepdims=True) + j * CB_BLK
        upd = m < best_m
        best_i = jnp.where(upd, am, best_i)
        best_m = jnp.where(upd, m, best_m)
    out_ref[...] = best_i.reshape(1, 1, ROW_BLK)


def _vq_indices(x, W_enc, codebook):
    out = pl.pallas_call(
        _vq_body,
        grid=(N_ROW,),
        in_specs=[
            pl.BlockSpec((1, CIN, H, W), lambda i: (i, 0, 0, 0)),
            pl.BlockSpec((PD, CODE_DIM), lambda i: (0, 0)),
            pl.BlockSpec((K, CODE_DIM), lambda i: (0, 0)),
        ],
        out_specs=pl.BlockSpec((1, 1, ROW_BLK), lambda i: (i, 0, 0)),
        out_shape=jax.ShapeDtypeStruct((N_ROW, 1, ROW_BLK), jnp.int32),
        scratch_shapes=[
            pltpu.VMEM((K, 1), jnp.float32),
        ],
        compiler_params=pltpu.CompilerParams(
            dimension_semantics=("arbitrary",),
        ),
    )(x, W_enc, codebook)
    return out.reshape(N)


NW = 32
B_PER_W = N // NW


def _sc_gather_body(cb_hbm, idx_hbm, out_hbm, idx_v, rows_v, sem):
    wid = lax.axis_index("s") * 2 + lax.axis_index("c")
    base = wid * B_PER_W
    pltpu.sync_copy(idx_hbm.at[pl.ds(base, B_PER_W)], idx_v)
    pltpu.async_copy(cb_hbm.at[idx_v], rows_v, sem).wait()
    pltpu.sync_copy(rows_v, out_hbm.at[pl.ds(base, B_PER_W)])


def _sc_gather(codebook, inds):
    k = functools.partial(
        pl.kernel,
        out_type=jax.ShapeDtypeStruct((N, CODE_DIM), jnp.float32),
        mesh=plsc.VectorSubcoreMesh(core_axis_name="c", subcore_axis_name="s"),
        scratch_types=[
            pltpu.VMEM((B_PER_W,), jnp.int32),
            pltpu.VMEM((B_PER_W, CODE_DIM), jnp.float32),
            pltpu.SemaphoreType.DMA,
        ],
    )(_sc_gather_body)
    return k(codebook, inds)


def _dec_body(hq_ref, wd_ref, out_ref):
    op = jnp.dot(hq_ref[...], wd_ref[...],
                 preferred_element_type=jnp.float32)
    op5 = op.reshape(GH, GW, CIN, PATCH, PATCH)
    out_ref[...] = op5.transpose(2, 0, 3, 1, 4).reshape(1, CIN, H, W)


def _decode(hq, W_dec):
    return pl.pallas_call(
        _dec_body,
        grid=(N_ROW,),
        in_specs=[
            pl.BlockSpec((ROW_BLK, CODE_DIM), lambda i: (i, 0)),
            pl.BlockSpec((CODE_DIM, PD), lambda i: (0, 0)),
        ],
        out_specs=pl.BlockSpec((1, CIN, H, W), lambda i: (i, 0, 0, 0)),
        out_shape=jax.ShapeDtypeStruct((B, CIN, H, W), jnp.float32),
        compiler_params=pltpu.CompilerParams(
            dimension_semantics=("parallel",),
        ),
    )(hq, W_dec)


def kernel(x, W_enc, codebook, W_dec):
    inds = _vq_indices(x, W_enc, codebook)
    hq = _sc_gather(codebook, inds)
    return _decode(hq, W_dec)
